# baseline (speedup 1.0000x reference)
import jax
import jax.numpy as jnp
from jax import lax
from jax.experimental import pallas as pl
from jax.experimental.pallas import tpu as pltpu
from jax.experimental.pallas import tpu_sc as plsc

F32 = jnp.float32
BF16 = jnp.bfloat16

NC = 2
NS = 16
NW = NC * NS

CE = 80


def _cdiv(a, b):
    return (a + b - 1) // b


def _gcd(a, b):
    while b:
        a, b = b, a % b
    return a


def _prep_body(x_ref, w1a_ref, w1b_ref, a_ref, b_ref):
    x = x_ref[...]
    a_ref[...] = jnp.dot(x, w1a_ref[...], preferred_element_type=F32)
    b_ref[...] = jnp.dot(x, w1b_ref[...], preferred_element_type=F32)


def _edge_body(pre_ref, dif_ref, eat_ref, w1d_ref, b1_ref, wr2_ref, w2_ref,
               b2_ref, w5_ref, b5_ref, m_ref, gd_ref):
    be = pre_ref.shape[0]
    xi = pre_ref[...]
    pp = dif_ref[:, :16]
    r2 = jnp.sum(pp * pp, axis=1, keepdims=True)
    eaw = lax.dot_general(eat_ref[...], w1d_ref[...], (((0,), (0,)), ((), ())),
                          preferred_element_type=F32)
    z1 = xi + eaw + r2 * wr2_ref[...] + b1_ref[...]
    m1 = jax.nn.silu(z1)
    z2 = jnp.dot(m1, w2_ref[...], preferred_element_type=F32) + b2_ref[...]
    m2 = jax.nn.silu(z2)
    gamma = jnp.dot(m2, w5_ref[...], preferred_element_type=F32) + b5_ref[...]
    col = lax.broadcasted_iota(jnp.int32, (1, 16), 1)
    degmark = (col == 3).astype(F32)
    m_ref[...] = m2
    gd_ref[...] = jnp.concatenate(
        [gamma * pp + degmark, jnp.zeros((be, 112), F32)], axis=1)


def _node_body(x_ref, p16_ref, *refs):
    (w3a_ref, w3b_ref, b3_ref, w4_ref, b4_ref, xo_ref, po_ref) = refs[-7:]
    parts = refs[:-7]
    nparts = len(parts) // 2
    accm = parts[0][...]
    accg = parts[nparts][...]
    for j in range(1, nparts):
        accm = accm + parts[j][...]
        accg = accg + parts[nparts + j][...]
    deg = jnp.maximum(accg[:, 3:4], 1.0)
    inv = 1.0 / deg
    msum = accm * inv
    z3 = (jnp.dot(x_ref[...], w3a_ref[...], preferred_element_type=F32)
          + jnp.dot(msum, w3b_ref[...], preferred_element_type=F32)
          + b3_ref[...])
    h3 = jax.nn.silu(z3)
    xo_ref[...] = jnp.dot(h3, w4_ref[...], preferred_element_type=F32) + b4_ref[...]
    po_ref[...] = p16_ref[...] + accg * inv


def _pipe(n_chunks, issue, wait, work):
    issue(0, 0)
    issue(1, 1)
    for p in range(2):
        wait(p, p)
        work(p, p, True)
        issue(p + 2, p)
    n_steady = (n_chunks - 2) // 2 - 1

    def pair(i, carry):
        for p in range(2):
            k = 2 + i * 2 + p
            wait(k, p)
            work(k, p, False)
            issue(k + 2, p)
        return carry

    lax.fori_loop(0, n_steady, pair, 0)
    for k in range(2 + 2 * n_steady, n_chunks):
        p = k % 2
        wait(k, p)
        work(k, p, False)
        if k + 2 < n_chunks:
            issue(k + 2, p)


def _sc_gather(a_t, b_t, dst2, src2, e_pad, width, tc_tiling, sub):
    e_per_w = e_pad // NW
    n_chunks = e_per_w // CE
    mesh = plsc.VectorSubcoreMesh(core_axis_name="c", subcore_axis_name="s")

    def body(a_hbm, b_hbm, d_hbm, s_hbm, pre_hbm, idx_d, idx_s,
             ba0, bb0, bo0, ba1, bb1, bo1, ga0, gb0, ga1, gb1, ws0, ws1):
        cid = lax.axis_index("c")
        sid = lax.axis_index("s")
        wid = sid * NC + cid
        ebase = wid * e_per_w
        sets = ((ba0, bb0, bo0, ga0, gb0, ws0), (ba1, bb1, bo1, ga1, gb1, ws1))

        pltpu.sync_copy(d_hbm.at[wid], idx_d)
        pltpu.sync_copy(s_hbm.at[wid], idx_s)

        def issue(k, p):
            ba, bb, _, ga, gb, _ = sets[p]
            pltpu.async_copy(a_hbm.at[idx_d.at[k]], ba, ga)
            pltpu.async_copy(b_hbm.at[idx_s.at[k]], bb, gb)

        def wait(k, p):
            ba, bb, _, ga, gb, _ = sets[p]
            pltpu.make_async_copy(a_hbm.at[idx_d.at[k]], ba, ga).wait()
            pltpu.make_async_copy(b_hbm.at[idx_s.at[k]], bb, gb).wait()

        def out_slab(k):
            if sub:
                return pre_hbm.at[pl.ds(ebase + k * CE, CE), pl.ds(0, width)]
            return pre_hbm.at[pl.ds(ebase + k * CE, CE)]

        def work(k, p, first):
            ba, bb, bo, _, _, ws = sets[p]
            out = out_slab(k)
            if not first:
                pltpu.make_async_copy(bo, out, ws).wait()

            def row(i, c2):
                for h in range(width // 16):
                    sl = pl.ds(h * 16, 16)
                    if sub:
                        bo[i, sl] = ba[i, sl] - bb[i, sl]
                    else:
                        bo[i, sl] = ba[i, sl] + bb[i, sl]
                return c2

            lax.fori_loop(0, CE, row, 0, unroll=4)
            pltpu.async_copy(bo, out, ws)

        _pipe(n_chunks, issue, wait, work)
        for p in range(2):
            _, _, bo, _, _, ws = sets[p]
            k = n_chunks - 2 + p
            pltpu.make_async_copy(bo, out_slab(k), ws).wait()

    out_type = jax.ShapeDtypeStruct((e_pad, 128 if sub else width), F32)
    fn = pl.kernel(
        body,
        out_type=out_type,
        mesh=mesh,
        compiler_params=pltpu.CompilerParams(use_tc_tiling_on_sc=tc_tiling),
        scratch_types=[
            pltpu.VMEM((n_chunks, CE), jnp.int32),
            pltpu.VMEM((n_chunks, CE), jnp.int32),
        ] + [pltpu.VMEM((CE, width), F32)] * 6
          + [pltpu.SemaphoreType.DMA] * 6,
    )
    return fn(a_t, b_t, dst2, src2)


def _sc_scatter(vals, dst2, e_pad, n_pad, width, tc_tiling, packed=False):
    e_per_w = e_pad // NW
    n_chunks = e_per_w // CE
    npsc = n_pad // NS
    mesh = plsc.VectorSubcoreMesh(core_axis_name="c", subcore_axis_name="s")

    def body(v_hbm, d_hbm, out_hbm, idx, b0, b1, acc, sem0, sem1):
        cid = lax.axis_index("c")
        sid = lax.axis_index("s")
        wid = sid * NC + cid
        ebase = wid * e_per_w
        bufs = (b0, b1)
        sems = (sem0, sem1)

        pltpu.sync_copy(d_hbm.at[wid], idx)

        def zrow(i, carry):
            for t in range(width // 16):
                b0[i, pl.ds(t * 16, 16)] = jnp.zeros((16,), F32)
            return carry

        lax.fori_loop(0, CE, zrow, 0)
        for t in range(npsc // CE):
            pltpu.sync_copy(b0, acc.at[pl.ds(sid * npsc + t * CE, CE)])
        plsc.subcore_barrier()

        def src_slab(k):
            if packed:
                return v_hbm.at[pl.ds(ebase + k * CE, CE), pl.ds(0, width)]
            return v_hbm.at[pl.ds(ebase + k * CE, CE)]

        def issue(k, p):
            pltpu.async_copy(src_slab(k), bufs[p], sems[p])

        def wait(k, p):
            pltpu.make_async_copy(src_slab(k), bufs[p], sems[p]).wait()

        def work(k, p, first):
            pltpu.sync_copy(bufs[p], acc.at[idx.at[k]], add=True)

        _pipe(n_chunks, issue, wait, work)
        plsc.subcore_barrier()
        pltpu.sync_copy(acc.at[pl.ds(sid * npsc, npsc)],
                        out_hbm.at[cid, pl.ds(sid * npsc, npsc)])

    fn = pl.kernel(
        body,
        out_type=jax.ShapeDtypeStruct((NC, n_pad, width), F32),
        mesh=mesh,
        compiler_params=pltpu.CompilerParams(use_tc_tiling_on_sc=tc_tiling),
        scratch_types=[
            pltpu.VMEM((n_chunks, CE), jnp.int32),
            pltpu.VMEM((CE, width), F32),
            pltpu.VMEM((CE, width), F32),
            pltpu.VMEM_SHARED((n_pad, width), F32),
            pltpu.SemaphoreType.DMA,
            pltpu.SemaphoreType.DMA,
        ],
    )
    return fn(vals, dst2)


def kernel(x, pos, edge_index, edge_attr, W1, b1, W2, b2, W3, b3, W4, b4, W5, b5):
    n, d = x.shape
    e = edge_index.shape[1]
    ed = edge_attr.shape[1]
    h = W2.shape[1]

    bn = 2048
    be = 2560
    n_pad = _cdiv(n, bn) * bn
    e_pad = _cdiv(e, NW * CE) * (NW * CE)

    src = edge_index[0]
    dst = edge_index[1]
    x_pad = jnp.pad(x, ((0, n_pad - n), (0, 0)))
    p16 = jnp.pad(pos, ((0, n_pad - n), (0, 16 - pos.shape[1])))
    if e_pad != e:
        src = jnp.pad(src, (0, e_pad - e))
        dst = jnp.pad(dst, (0, e_pad - e), constant_values=n_pad - 1)
        edge_attr = jnp.pad(edge_attr, ((0, e_pad - e), (0, 0)))
    grain = NW * CE * (be // _gcd(be, NW * CE))
    half = _cdiv(e_pad // 2, grain) * grain
    splits = [(0, half), (half, e_pad - half)] if 0 < half < e_pad else [(0, e_pad)]

    w1a = W1[:d]
    w1b = W1[d:2 * d]
    wr2 = W1[2 * d:2 * d + 1]
    w1d = W1[2 * d + 1:]
    b1r = b1.reshape(1, h)
    b2r = b2.reshape(1, h)
    b3r = b3.reshape(1, h)
    b4r = b4.reshape(1, d)
    b5r = b5.reshape(1, 1)
    w3a = W3[:d]
    w3b = W3[d:]

    full = lambda a: pl.BlockSpec(a.shape, lambda i: (0,) * a.ndim)

    a_t, b_t = pl.pallas_call(
        _prep_body,
        grid=(n_pad // bn,),
        in_specs=[
            pl.BlockSpec((bn, d), lambda i: (i, 0)),
            full(w1a), full(w1b),
        ],
        out_specs=[pl.BlockSpec((bn, d), lambda i: (i, 0))] * 2,
        out_shape=[jax.ShapeDtypeStruct((n_pad, d), F32)] * 2,
    )(x_pad, w1a, w1b)

    eat = edge_attr.T
    pms, pgs = [], []
    for e0, esz in splits:
        dst2 = lax.dynamic_slice_in_dim(dst, e0, esz).reshape(
            NW, esz // NW // CE, CE)
        src2 = lax.dynamic_slice_in_dim(src, e0, esz).reshape(
            NW, esz // NW // CE, CE)
        pre = _sc_gather(a_t, b_t, dst2, src2, esz, d, True, False)
        dif = _sc_gather(p16, p16, dst2, src2, esz, 16, False, True)
        eat_s = lax.dynamic_slice_in_dim(eat, e0, esz, axis=1)
        m_ij, gd2 = pl.pallas_call(
            _edge_body,
            grid=(esz // be,),
            in_specs=[
                pl.BlockSpec((be, d), lambda i: (i, 0)),
                pl.BlockSpec((be, 128), lambda i: (i, 0)),
                pl.BlockSpec((ed, be), lambda i: (0, i)),
                full(w1d), full(b1r), full(wr2), full(W2), full(b2r),
                full(W5), full(b5r),
            ],
            out_specs=[
                pl.BlockSpec((be, 128), lambda i: (i, 0)),
                pl.BlockSpec((be, 128), lambda i: (i, 0)),
            ],
            out_shape=[
                jax.ShapeDtypeStruct((esz, 128), F32),
                jax.ShapeDtypeStruct((esz, 128), F32),
            ],
        )(pre, dif, eat_s, w1d, b1r, wr2, W2, b2r, W5, b5r)
        pms.append(_sc_scatter(m_ij, dst2, esz, n_pad, 128, True))
        pgs.append(_sc_scatter(gd2, dst2, esz, n_pad, 16, False, packed=True))

    pm_parts = [p[c] for p in pms for c in range(NC)]
    pg_parts = [p[c] for p in pgs for c in range(NC)]

    nparts = len(pm_parts)
    xo, po16 = pl.pallas_call(
        _node_body,
        grid=(n_pad // bn,),
        in_specs=[
            pl.BlockSpec((bn, d), lambda i: (i, 0)),
            pl.BlockSpec((bn, 16), lambda i: (i, 0)),
        ] + [pl.BlockSpec((bn, 128), lambda i: (i, 0))] * nparts
          + [pl.BlockSpec((bn, 16), lambda i: (i, 0))] * nparts
          + [full(w3a), full(w3b), full(b3r), full(W4), full(b4r)],
        out_specs=[
            pl.BlockSpec((bn, d), lambda i: (i, 0)),
            pl.BlockSpec((bn, 16), lambda i: (i, 0)),
        ],
        out_shape=[
            jax.ShapeDtypeStruct((n_pad, d), F32),
            jax.ShapeDtypeStruct((n_pad, 16), F32),
        ],
    )(x_pad, p16, *pm_parts, *pg_parts, w3a, w3b, b3r, W4, b4r)

    return (xo[:n], po16[:n, :pos.shape[1]])


# speedup vs baseline: 1.0060x; 1.0060x over previous
import jax
import jax.numpy as jnp
from jax import lax
from jax.experimental import pallas as pl
from jax.experimental.pallas import tpu as pltpu
from jax.experimental.pallas import tpu_sc as plsc

F32 = jnp.float32
BF16 = jnp.bfloat16

NC = 2
NS = 16
NW = NC * NS

CE = 80


def _cdiv(a, b):
    return (a + b - 1) // b


def _gcd(a, b):
    while b:
        a, b = b, a % b
    return a


def _prep_body(x_ref, w1a_ref, w1b_ref, a_ref, b_ref):
    x = x_ref[...]
    a_ref[...] = jnp.dot(x, w1a_ref[...], preferred_element_type=F32)
    b_ref[...] = jnp.dot(x, w1b_ref[...], preferred_element_type=F32)


def _edge_body(pre_ref, dif_ref, eat_ref, w1d_ref, b1_ref, wr2_ref, w2_ref,
               b2_ref, w5_ref, b5_ref, m_ref, gd_ref):
    be = pre_ref.shape[0]
    xi = pre_ref[...]
    pp = dif_ref[:, :16]
    r2 = jnp.sum(pp * pp, axis=1, keepdims=True)
    eaw = lax.dot_general(eat_ref[...], w1d_ref[...], (((0,), (0,)), ((), ())),
                          preferred_element_type=F32)
    z1 = xi + eaw + r2 * wr2_ref[...] + b1_ref[...]
    m1 = jax.nn.silu(z1)
    z2 = jnp.dot(m1, w2_ref[...], preferred_element_type=F32) + b2_ref[...]
    m2 = jax.nn.silu(z2)
    gamma = jnp.dot(m2, w5_ref[...], preferred_element_type=F32) + b5_ref[...]
    col = lax.broadcasted_iota(jnp.int32, (1, 16), 1)
    degmark = (col == 3).astype(F32)
    m_ref[...] = m2
    gd_ref[...] = jnp.concatenate(
        [gamma * pp + degmark, jnp.zeros((be, 112), F32)], axis=1)


def _node_body(x_ref, p16_ref, *refs):
    (w3a_ref, w3b_ref, b3_ref, w4_ref, b4_ref, xo_ref, po_ref) = refs[-7:]
    parts = refs[:-7]
    nparts = len(parts) // 2
    accm = parts[0][...]
    accg = parts[nparts][...]
    for j in range(1, nparts):
        accm = accm + parts[j][...]
        accg = accg + parts[nparts + j][...]
    deg = jnp.maximum(accg[:, 3:4], 1.0)
    inv = 1.0 / deg
    msum = accm * inv
    z3 = (jnp.dot(x_ref[...], w3a_ref[...], preferred_element_type=F32)
          + jnp.dot(msum, w3b_ref[...], preferred_element_type=F32)
          + b3_ref[...])
    h3 = jax.nn.silu(z3)
    xo_ref[...] = jnp.dot(h3, w4_ref[...], preferred_element_type=F32) + b4_ref[...]
    po_ref[...] = p16_ref[...] + accg * inv


def _pipe(n_chunks, issue, wait, work):
    issue(0, 0)
    issue(1, 1)
    for p in range(2):
        wait(p, p)
        work(p, p, True)
        issue(p + 2, p)
    n_steady = (n_chunks - 2) // 2 - 1

    def pair(i, carry):
        for p in range(2):
            k = 2 + i * 2 + p
            wait(k, p)
            work(k, p, False)
            issue(k + 2, p)
        return carry

    lax.fori_loop(0, n_steady, pair, 0)
    for k in range(2 + 2 * n_steady, n_chunks):
        p = k % 2
        wait(k, p)
        work(k, p, False)
        if k + 2 < n_chunks:
            issue(k + 2, p)


def _sc_gather(a_t, b_t, dst2, src2, e_pad, width, tc_tiling, sub):
    e_per_w = e_pad // NW
    n_chunks = e_per_w // CE
    mesh = plsc.VectorSubcoreMesh(core_axis_name="c", subcore_axis_name="s")

    def body(a_hbm, b_hbm, d_hbm, s_hbm, pre_hbm, idx_d, idx_s,
             ba0, bb0, bo0, ba1, bb1, bo1, ga0, gb0, ga1, gb1, ws0, ws1):
        cid = lax.axis_index("c")
        sid = lax.axis_index("s")
        wid = sid * NC + cid
        ebase = wid * e_per_w
        sets = ((ba0, bb0, bo0, ga0, gb0, ws0), (ba1, bb1, bo1, ga1, gb1, ws1))

        pltpu.sync_copy(d_hbm.at[wid], idx_d)
        pltpu.sync_copy(s_hbm.at[wid], idx_s)

        def issue(k, p):
            ba, bb, _, ga, gb, _ = sets[p]
            pltpu.async_copy(a_hbm.at[idx_d.at[k]], ba, ga)
            pltpu.async_copy(b_hbm.at[idx_s.at[k]], bb, gb)

        def wait(k, p):
            ba, bb, _, ga, gb, _ = sets[p]
            pltpu.make_async_copy(a_hbm.at[idx_d.at[k]], ba, ga).wait()
            pltpu.make_async_copy(b_hbm.at[idx_s.at[k]], bb, gb).wait()

        def out_slab(k):
            if sub:
                return pre_hbm.at[pl.ds(ebase + k * CE, CE), pl.ds(0, width)]
            return pre_hbm.at[pl.ds(ebase + k * CE, CE)]

        def work(k, p, first):
            ba, bb, bo, _, _, ws = sets[p]
            out = out_slab(k)
            if not first:
                pltpu.make_async_copy(bo, out, ws).wait()

            def row(i, c2):
                for h in range(width // 16):
                    sl = pl.ds(h * 16, 16)
                    if sub:
                        bo[i, sl] = ba[i, sl] - bb[i, sl]
                    else:
                        bo[i, sl] = ba[i, sl] + bb[i, sl]
                return c2

            lax.fori_loop(0, CE, row, 0, unroll=2)
            pltpu.async_copy(bo, out, ws)

        _pipe(n_chunks, issue, wait, work)
        for p in range(2):
            _, _, bo, _, _, ws = sets[p]
            k = n_chunks - 2 + p
            pltpu.make_async_copy(bo, out_slab(k), ws).wait()

    out_type = jax.ShapeDtypeStruct((e_pad, 128 if sub else width), F32)
    fn = pl.kernel(
        body,
        out_type=out_type,
        mesh=mesh,
        compiler_params=pltpu.CompilerParams(use_tc_tiling_on_sc=tc_tiling),
        scratch_types=[
            pltpu.VMEM((n_chunks, CE), jnp.int32),
            pltpu.VMEM((n_chunks, CE), jnp.int32),
        ] + [pltpu.VMEM((CE, width), F32)] * 6
          + [pltpu.SemaphoreType.DMA] * 6,
    )
    return fn(a_t, b_t, dst2, src2)


def _sc_scatter(vals, dst2, e_pad, n_pad, width, tc_tiling, packed=False):
    e_per_w = e_pad // NW
    n_chunks = e_per_w // CE
    npsc = n_pad // NS
    mesh = plsc.VectorSubcoreMesh(core_axis_name="c", subcore_axis_name="s")

    def body(v_hbm, d_hbm, out_hbm, idx, b0, b1, acc, sem0, sem1):
        cid = lax.axis_index("c")
        sid = lax.axis_index("s")
        wid = sid * NC + cid
        ebase = wid * e_per_w
        bufs = (b0, b1)
        sems = (sem0, sem1)

        pltpu.sync_copy(d_hbm.at[wid], idx)

        def zrow(i, carry):
            for t in range(width // 16):
                b0[i, pl.ds(t * 16, 16)] = jnp.zeros((16,), F32)
            return carry

        lax.fori_loop(0, CE, zrow, 0)
        for t in range(npsc // CE):
            pltpu.sync_copy(b0, acc.at[pl.ds(sid * npsc + t * CE, CE)])
        plsc.subcore_barrier()

        def src_slab(k):
            if packed:
                return v_hbm.at[pl.ds(ebase + k * CE, CE), pl.ds(0, width)]
            return v_hbm.at[pl.ds(ebase + k * CE, CE)]

        def issue(k, p):
            pltpu.async_copy(src_slab(k), bufs[p], sems[p])

        def wait(k, p):
            pltpu.make_async_copy(src_slab(k), bufs[p], sems[p]).wait()

        def work(k, p, first):
            pltpu.sync_copy(bufs[p], acc.at[idx.at[k]], add=True)

        _pipe(n_chunks, issue, wait, work)
        plsc.subcore_barrier()
        pltpu.sync_copy(acc.at[pl.ds(sid * npsc, npsc)],
                        out_hbm.at[cid, pl.ds(sid * npsc, npsc)])

    fn = pl.kernel(
        body,
        out_type=jax.ShapeDtypeStruct((NC, n_pad, width), F32),
        mesh=mesh,
        compiler_params=pltpu.CompilerParams(use_tc_tiling_on_sc=tc_tiling),
        scratch_types=[
            pltpu.VMEM((n_chunks, CE), jnp.int32),
            pltpu.VMEM((CE, width), F32),
            pltpu.VMEM((CE, width), F32),
            pltpu.VMEM_SHARED((n_pad, width), F32),
            pltpu.SemaphoreType.DMA,
            pltpu.SemaphoreType.DMA,
        ],
    )
    return fn(vals, dst2)


def kernel(x, pos, edge_index, edge_attr, W1, b1, W2, b2, W3, b3, W4, b4, W5, b5):
    n, d = x.shape
    e = edge_index.shape[1]
    ed = edge_attr.shape[1]
    h = W2.shape[1]

    bn = 2048
    be = 2560
    n_pad = _cdiv(n, bn) * bn
    e_pad = _cdiv(e, NW * CE) * (NW * CE)

    src = edge_index[0]
    dst = edge_index[1]
    x_pad = jnp.pad(x, ((0, n_pad - n), (0, 0)))
    p16 = jnp.pad(pos, ((0, n_pad - n), (0, 16 - pos.shape[1])))
    if e_pad != e:
        src = jnp.pad(src, (0, e_pad - e))
        dst = jnp.pad(dst, (0, e_pad - e), constant_values=n_pad - 1)
        edge_attr = jnp.pad(edge_attr, ((0, e_pad - e), (0, 0)))
    grain = NW * CE * (be // _gcd(be, NW * CE))
    nslices = 3
    splits = []
    off = 0
    for s in range(nslices):
        sz = _cdiv((e_pad - off) // (nslices - s), grain) * grain
        sz = min(sz, e_pad - off)
        if sz > 0:
            splits.append((off, sz))
            off += sz
    if off < e_pad:
        splits[-1] = (splits[-1][0], splits[-1][1] + e_pad - off)

    w1a = W1[:d]
    w1b = W1[d:2 * d]
    wr2 = W1[2 * d:2 * d + 1]
    w1d = W1[2 * d + 1:]
    b1r = b1.reshape(1, h)
    b2r = b2.reshape(1, h)
    b3r = b3.reshape(1, h)
    b4r = b4.reshape(1, d)
    b5r = b5.reshape(1, 1)
    w3a = W3[:d]
    w3b = W3[d:]

    full = lambda a: pl.BlockSpec(a.shape, lambda i: (0,) * a.ndim)

    a_t, b_t = pl.pallas_call(
        _prep_body,
        grid=(n_pad // bn,),
        in_specs=[
            pl.BlockSpec((bn, d), lambda i: (i, 0)),
            full(w1a), full(w1b),
        ],
        out_specs=[pl.BlockSpec((bn, d), lambda i: (i, 0))] * 2,
        out_shape=[jax.ShapeDtypeStruct((n_pad, d), F32)] * 2,
    )(x_pad, w1a, w1b)

    eat = edge_attr.T
    pms, pgs = [], []
    for e0, esz in splits:
        dst2 = lax.dynamic_slice_in_dim(dst, e0, esz).reshape(
            NW, esz // NW // CE, CE)
        src2 = lax.dynamic_slice_in_dim(src, e0, esz).reshape(
            NW, esz // NW // CE, CE)
        pre = _sc_gather(a_t, b_t, dst2, src2, esz, d, True, False)
        dif = _sc_gather(p16, p16, dst2, src2, esz, 16, False, True)
        eat_s = lax.dynamic_slice_in_dim(eat, e0, esz, axis=1)
        m_ij, gd2 = pl.pallas_call(
            _edge_body,
            grid=(esz // be,),
            in_specs=[
                pl.BlockSpec((be, d), lambda i: (i, 0)),
                pl.BlockSpec((be, 128), lambda i: (i, 0)),
                pl.BlockSpec((ed, be), lambda i: (0, i)),
                full(w1d), full(b1r), full(wr2), full(W2), full(b2r),
                full(W5), full(b5r),
            ],
            out_specs=[
                pl.BlockSpec((be, 128), lambda i: (i, 0)),
                pl.BlockSpec((be, 128), lambda i: (i, 0)),
            ],
            out_shape=[
                jax.ShapeDtypeStruct((esz, 128), F32),
                jax.ShapeDtypeStruct((esz, 128), F32),
            ],
        )(pre, dif, eat_s, w1d, b1r, wr2, W2, b2r, W5, b5r)
        pms.append(_sc_scatter(m_ij, dst2, esz, n_pad, 128, True))
        pgs.append(_sc_scatter(gd2, dst2, esz, n_pad, 16, False, packed=True))

    pm_parts = [p[c] for p in pms for c in range(NC)]
    pg_parts = [p[c] for p in pgs for c in range(NC)]

    nparts = len(pm_parts)
    xo, po16 = pl.pallas_call(
        _node_body,
        grid=(n_pad // bn,),
        in_specs=[
            pl.BlockSpec((bn, d), lambda i: (i, 0)),
            pl.BlockSpec((bn, 16), lambda i: (i, 0)),
        ] + [pl.BlockSpec((bn, 128), lambda i: (i, 0))] * nparts
          + [pl.BlockSpec((bn, 16), lambda i: (i, 0))] * nparts
          + [full(w3a), full(w3b), full(b3r), full(W4), full(b4r)],
        out_specs=[
            pl.BlockSpec((bn, d), lambda i: (i, 0)),
            pl.BlockSpec((bn, 16), lambda i: (i, 0)),
        ],
        out_shape=[
            jax.ShapeDtypeStruct((n_pad, d), F32),
            jax.ShapeDtypeStruct((n_pad, 16), F32),
        ],
    )(x_pad, p16, *pm_parts, *pg_parts, w3a, w3b, b3r, W4, b4r)

    return (xo[:n], po16[:n, :pos.shape[1]])


# speedup vs baseline: 1.0065x; 1.0005x over previous
import jax
import jax.numpy as jnp
from jax import lax
from jax.experimental import pallas as pl
from jax.experimental.pallas import tpu as pltpu
from jax.experimental.pallas import tpu_sc as plsc

F32 = jnp.float32

NC = 2
NS = 16
NW = NC * NS

CE = 80


def _cdiv(a, b):
    return (a + b - 1) // b


def _gcd(a, b):
    while b:
        a, b = b, a % b
    return a


def _prep_body(x_ref, w1a_ref, w1b_ref, a_ref, b_ref):
    x = x_ref[...]
    a_ref[...] = jnp.dot(x, w1a_ref[...], preferred_element_type=F32)
    b_ref[...] = jnp.dot(x, w1b_ref[...], preferred_element_type=F32)


def _edge_body(pre_ref, dif_ref, eat_ref, w1d_ref, b1_ref, wr2_ref, w2_ref,
               b2_ref, w5_ref, b5_ref, m_ref, gd_ref):
    be = pre_ref.shape[0]
    xi = pre_ref[...]
    pp = dif_ref[:, :16]
    r2 = jnp.sum(pp * pp, axis=1, keepdims=True)
    eaw = lax.dot_general(eat_ref[...], w1d_ref[...], (((0,), (0,)), ((), ())),
                          preferred_element_type=F32)
    z1 = xi + eaw + r2 * wr2_ref[...] + b1_ref[...]
    m1 = jax.nn.silu(z1)
    z2 = jnp.dot(m1, w2_ref[...], preferred_element_type=F32) + b2_ref[...]
    m2 = jax.nn.silu(z2)
    gamma = jnp.dot(m2, w5_ref[...], preferred_element_type=F32) + b5_ref[...]
    col = lax.broadcasted_iota(jnp.int32, (1, 16), 1)
    degmark = (col == 3).astype(F32)
    m_ref[...] = m2
    gd_ref[...] = jnp.concatenate(
        [gamma * pp + degmark, jnp.zeros((be, 112), F32)], axis=1)


def _node_body(x_ref, p16_ref, *refs):
    (w3a_ref, w3b_ref, b3_ref, w4_ref, b4_ref, xo_ref, po_ref) = refs[-7:]
    parts = refs[:-7]
    nparts = len(parts) // 2
    accm = parts[0][...]
    accg = parts[nparts][...]
    for j in range(1, nparts):
        accm = accm + parts[j][...]
        accg = accg + parts[nparts + j][...]
    deg = jnp.maximum(accg[:, 3:4], 1.0)
    inv = 1.0 / deg
    msum = accm * inv
    z3 = (jnp.dot(x_ref[...], w3a_ref[...], preferred_element_type=F32)
          + jnp.dot(msum, w3b_ref[...], preferred_element_type=F32)
          + b3_ref[...])
    h3 = jax.nn.silu(z3)
    xo_ref[...] = jnp.dot(h3, w4_ref[...], preferred_element_type=F32) + b4_ref[...]
    po_ref[...] = p16_ref[...] + accg * inv


def _pipe(n_chunks, issue, wait, work):
    issue(0, 0)
    issue(1, 1)
    for p in range(2):
        wait(p, p)
        work(p, p, True)
        issue(p + 2, p)
    n_steady = (n_chunks - 2) // 2 - 1

    def pair(i, carry):
        for p in range(2):
            k = 2 + i * 2 + p
            wait(k, p)
            work(k, p, False)
            issue(k + 2, p)
        return carry

    lax.fori_loop(0, n_steady, pair, 0)
    for k in range(2 + 2 * n_steady, n_chunks):
        p = k % 2
        wait(k, p)
        work(k, p, False)
        if k + 2 < n_chunks:
            issue(k + 2, p)


def _sc_gather(a_t, b_t, dst2, src2, e_pad, width, tc_tiling, sub):
    e_per_w = e_pad // NW
    n_chunks = e_per_w // CE
    mesh = plsc.VectorSubcoreMesh(core_axis_name="c", subcore_axis_name="s")

    def body(a_hbm, b_hbm, d_hbm, s_hbm, pre_hbm, idx_d, idx_s,
             ba0, bb0, bo0, ba1, bb1, bo1, ga0, gb0, ga1, gb1, ws0, ws1):
        cid = lax.axis_index("c")
        sid = lax.axis_index("s")
        wid = sid * NC + cid
        ebase = wid * e_per_w
        sets = ((ba0, bb0, bo0, ga0, gb0, ws0), (ba1, bb1, bo1, ga1, gb1, ws1))

        pltpu.sync_copy(d_hbm.at[wid], idx_d)
        pltpu.sync_copy(s_hbm.at[wid], idx_s)

        def issue(k, p):
            ba, bb, _, ga, gb, _ = sets[p]
            pltpu.async_copy(a_hbm.at[idx_d.at[k]], ba, ga)
            pltpu.async_copy(b_hbm.at[idx_s.at[k]], bb, gb)

        def wait(k, p):
            ba, bb, _, ga, gb, _ = sets[p]
            pltpu.make_async_copy(a_hbm.at[idx_d.at[k]], ba, ga).wait()
            pltpu.make_async_copy(b_hbm.at[idx_s.at[k]], bb, gb).wait()

        def out_slab(k):
            if sub:
                return pre_hbm.at[pl.ds(ebase + k * CE, CE), pl.ds(0, width)]
            return pre_hbm.at[pl.ds(ebase + k * CE, CE)]

        def work(k, p, first):
            ba, bb, bo, _, _, ws = sets[p]
            out = out_slab(k)
            if not first:
                pltpu.make_async_copy(bo, out, ws).wait()

            def row(i, c2):
                for h in range(width // 16):
                    sl = pl.ds(h * 16, 16)
                    if sub:
                        bo[i, sl] = ba[i, sl] - bb[i, sl]
                    else:
                        bo[i, sl] = ba[i, sl] + bb[i, sl]
                return c2

            lax.fori_loop(0, CE, row, 0, unroll=2)
            pltpu.async_copy(bo, out, ws)

        _pipe(n_chunks, issue, wait, work)
        for p in range(2):
            _, _, bo, _, _, ws = sets[p]
            k = n_chunks - 2 + p
            pltpu.make_async_copy(bo, out_slab(k), ws).wait()

    out_type = jax.ShapeDtypeStruct((e_pad, 128 if sub else width), F32)
    fn = pl.kernel(
        body,
        out_type=out_type,
        mesh=mesh,
        compiler_params=pltpu.CompilerParams(use_tc_tiling_on_sc=tc_tiling),
        scratch_types=[
            pltpu.VMEM((n_chunks, CE), jnp.int32),
            pltpu.VMEM((n_chunks, CE), jnp.int32),
        ] + [pltpu.VMEM((CE, width), F32)] * 6
          + [pltpu.SemaphoreType.DMA] * 6,
    )
    return fn(a_t, b_t, dst2, src2)


def _sc_scatter(vals, dst2, e_pad, n_pad, width, tc_tiling, packed=False):
    e_per_w = e_pad // NW
    n_chunks = e_per_w // CE
    npsc = n_pad // NS
    mesh = plsc.VectorSubcoreMesh(core_axis_name="c", subcore_axis_name="s")

    def body(v_hbm, d_hbm, out_hbm, idx, b0, b1, acc, sem0, sem1):
        cid = lax.axis_index("c")
        sid = lax.axis_index("s")
        wid = sid * NC + cid
        ebase = wid * e_per_w
        bufs = (b0, b1)
        sems = (sem0, sem1)

        pltpu.sync_copy(d_hbm.at[wid], idx)

        def zrow(i, carry):
            for t in range(width // 16):
                b0[i, pl.ds(t * 16, 16)] = jnp.zeros((16,), F32)
            return carry

        lax.fori_loop(0, CE, zrow, 0)
        for t in range(npsc // CE):
            pltpu.sync_copy(b0, acc.at[pl.ds(sid * npsc + t * CE, CE)])
        plsc.subcore_barrier()

        def src_slab(k):
            if packed:
                return v_hbm.at[pl.ds(ebase + k * CE, CE), pl.ds(0, width)]
            return v_hbm.at[pl.ds(ebase + k * CE, CE)]

        def issue(k, p):
            pltpu.async_copy(src_slab(k), bufs[p], sems[p])

        def wait(k, p):
            pltpu.make_async_copy(src_slab(k), bufs[p], sems[p]).wait()

        def work(k, p, first):
            pltpu.sync_copy(bufs[p], acc.at[idx.at[k]], add=True)

        _pipe(n_chunks, issue, wait, work)
        plsc.subcore_barrier()
        pltpu.sync_copy(acc.at[pl.ds(sid * npsc, npsc)],
                        out_hbm.at[cid, pl.ds(sid * npsc, npsc)])

    fn = pl.kernel(
        body,
        out_type=jax.ShapeDtypeStruct((NC, n_pad, width), F32),
        mesh=mesh,
        compiler_params=pltpu.CompilerParams(use_tc_tiling_on_sc=tc_tiling),
        scratch_types=[
            pltpu.VMEM((n_chunks, CE), jnp.int32),
            pltpu.VMEM((CE, width), F32),
            pltpu.VMEM((CE, width), F32),
            pltpu.VMEM_SHARED((n_pad, width), F32),
            pltpu.SemaphoreType.DMA,
            pltpu.SemaphoreType.DMA,
        ],
    )
    return fn(vals, dst2)


def kernel(x, pos, edge_index, edge_attr, W1, b1, W2, b2, W3, b3, W4, b4, W5, b5):
    n, d = x.shape
    e = edge_index.shape[1]
    ed = edge_attr.shape[1]
    h = W2.shape[1]

    bn = 2048
    be = 2560
    n_pad = _cdiv(n, bn) * bn
    e_pad = _cdiv(e, NW * CE) * (NW * CE)

    src = edge_index[0]
    dst = edge_index[1]
    x_pad = jnp.pad(x, ((0, n_pad - n), (0, 0)))
    p16 = jnp.pad(pos, ((0, n_pad - n), (0, 16 - pos.shape[1])))
    if e_pad != e:
        src = jnp.pad(src, (0, e_pad - e))
        dst = jnp.pad(dst, (0, e_pad - e), constant_values=n_pad - 1)
        edge_attr = jnp.pad(edge_attr, ((0, e_pad - e), (0, 0)))
    grain = NW * CE * (be // _gcd(be, NW * CE))
    nslices = 2
    splits = []
    off = 0
    for s in range(nslices):
        sz = _cdiv((e_pad - off) // (nslices - s), grain) * grain
        sz = min(sz, e_pad - off)
        if sz > 0:
            splits.append((off, sz))
            off += sz
    if off < e_pad:
        splits[-1] = (splits[-1][0], splits[-1][1] + e_pad - off)

    w1a = W1[:d]
    w1b = W1[d:2 * d]
    wr2 = W1[2 * d:2 * d + 1]
    w1d = W1[2 * d + 1:]
    b1r = b1.reshape(1, h)
    b2r = b2.reshape(1, h)
    b3r = b3.reshape(1, h)
    b4r = b4.reshape(1, d)
    b5r = b5.reshape(1, 1)
    w3a = W3[:d]
    w3b = W3[d:]

    full = lambda a: pl.BlockSpec(a.shape, lambda i: (0,) * a.ndim)

    a_t, b_t = pl.pallas_call(
        _prep_body,
        grid=(n_pad // bn,),
        in_specs=[
            pl.BlockSpec((bn, d), lambda i: (i, 0)),
            full(w1a), full(w1b),
        ],
        out_specs=[pl.BlockSpec((bn, d), lambda i: (i, 0))] * 2,
        out_shape=[jax.ShapeDtypeStruct((n_pad, d), F32)] * 2,
    )(x_pad, w1a, w1b)

    eat = edge_attr.T
    pms, pgs = [], []
    for e0, esz in splits:
        dst2 = lax.dynamic_slice_in_dim(dst, e0, esz).reshape(
            NW, esz // NW // CE, CE)
        src2 = lax.dynamic_slice_in_dim(src, e0, esz).reshape(
            NW, esz // NW // CE, CE)
        pre = _sc_gather(a_t, b_t, dst2, src2, esz, d, True, False)
        dif = _sc_gather(p16, p16, dst2, src2, esz, 16, False, True)
        eat_s = lax.dynamic_slice_in_dim(eat, e0, esz, axis=1)
        m_ij, gd2 = pl.pallas_call(
            _edge_body,
            grid=(esz // be,),
            in_specs=[
                pl.BlockSpec((be, d), lambda i: (i, 0)),
                pl.BlockSpec((be, 128), lambda i: (i, 0)),
                pl.BlockSpec((ed, be), lambda i: (0, i)),
                full(w1d), full(b1r), full(wr2), full(W2), full(b2r),
                full(W5), full(b5r),
            ],
            out_specs=[
                pl.BlockSpec((be, 128), lambda i: (i, 0)),
                pl.BlockSpec((be, 128), lambda i: (i, 0)),
            ],
            out_shape=[
                jax.ShapeDtypeStruct((esz, 128), F32),
                jax.ShapeDtypeStruct((esz, 128), F32),
            ],
        )(pre, dif, eat_s, w1d, b1r, wr2, W2, b2r, W5, b5r)
        pms.append(_sc_scatter(m_ij, dst2, esz, n_pad, 128, True))
        pgs.append(_sc_scatter(gd2, dst2, esz, n_pad, 16, False, packed=True))

    pm_parts = [p[c] for p in pms for c in range(NC)]
    pg_parts = [p[c] for p in pgs for c in range(NC)]

    nparts = len(pm_parts)
    xo, po16 = pl.pallas_call(
        _node_body,
        grid=(n_pad // bn,),
        in_specs=[
            pl.BlockSpec((bn, d), lambda i: (i, 0)),
            pl.BlockSpec((bn, 16), lambda i: (i, 0)),
        ] + [pl.BlockSpec((bn, 128), lambda i: (i, 0))] * nparts
          + [pl.BlockSpec((bn, 16), lambda i: (i, 0))] * nparts
          + [full(w3a), full(w3b), full(b3r), full(W4), full(b4r)],
        out_specs=[
            pl.BlockSpec((bn, d), lambda i: (i, 0)),
            pl.BlockSpec((bn, 16), lambda i: (i, 0)),
        ],
        out_shape=[
            jax.ShapeDtypeStruct((n_pad, d), F32),
            jax.ShapeDtypeStruct((n_pad, 16), F32),
        ],
    )(x_pad, p16, *pm_parts, *pg_parts, w3a, w3b, b3r, W4, b4r)

    return (xo[:n], po16[:n, :pos.shape[1]])
